# baseline (speedup 1.0000x reference)
import jax
import jax.numpy as jnp
from jax import lax
from jax.experimental import pallas as pl
from jax.experimental.pallas import tpu as pltpu
from jax.experimental.pallas import tpu_sc as plsc

NUM_CORES = 2
NUM_SUBCORES = 16
NUM_WORKERS = NUM_CORES * NUM_SUBCORES

B_PER_W = 512
CHUNK = 256
N_CHUNKS = B_PER_W // CHUNK
DIM = 64
LANES = 16


def _glove_body(idx_hbm, wi_hbm, wj_hbm, out_hbm,
                idx_sh, idx_s, rows_a, rows_b, sem_a, sem_b):
    wid = lax.axis_index("s") * NUM_CORES + lax.axis_index("c")
    base = wid * B_PER_W
    pltpu.sync_copy(idx_hbm.at[pl.ds(base, B_PER_W)], idx_sh.at[wid])
    pltpu.sync_copy(idx_sh.at[wid], idx_s)

    for ch in range(N_CHUNKS):
        def fire(i, _):
            r = idx_s[ch * CHUNK + i]
            pltpu.async_copy(wi_hbm.at[r], rows_a.at[i], sem_a)
            pltpu.async_copy(wj_hbm.at[r], rows_b.at[i], sem_b)
            return 0

        lax.fori_loop(0, CHUNK, fire, 0)

        dummy = out_hbm.at[pl.ds(base + ch * CHUNK, CHUNK)]
        pltpu.make_async_copy(dummy, rows_a, sem_a).wait()
        pltpu.make_async_copy(dummy, rows_b, sem_b).wait()

        def add(i, _):
            for c in range(DIM // LANES):
                sl = pl.ds(c * LANES, LANES)
                rows_a[i, sl] = rows_a[i, sl] + rows_b[i, sl]
            return 0

        lax.fori_loop(0, CHUNK, add, 0)

        pltpu.sync_copy(rows_a, out_hbm.at[pl.ds(base + ch * CHUNK, CHUNK)])


@jax.jit
def kernel(indices, wi, wj):
    batch = indices.shape[0]
    num_emb, dim = wi.shape
    assert batch == NUM_WORKERS * B_PER_W and dim == DIM

    mesh = plsc.VectorSubcoreMesh(core_axis_name="c", subcore_axis_name="s")
    run = pl.kernel(
        _glove_body,
        out_type=jax.ShapeDtypeStruct((batch, dim), jnp.float32),
        mesh=mesh,
        scratch_types=[
            pltpu.VMEM_SHARED((NUM_WORKERS, B_PER_W), jnp.int32),
            pltpu.SMEM((B_PER_W,), jnp.int32),
            pltpu.VMEM((CHUNK, DIM), jnp.float32),
            pltpu.VMEM((CHUNK, DIM), jnp.float32),
            pltpu.SemaphoreType.DMA,
            pltpu.SemaphoreType.DMA,
        ],
    )
    return run(indices.astype(jnp.int32), wi, wj)


# speedup vs baseline: 1.5198x; 1.5198x over previous
import jax
import jax.numpy as jnp
from jax import lax
from jax.experimental import pallas as pl
from jax.experimental.pallas import tpu as pltpu
from jax.experimental.pallas import tpu_sc as plsc

NUM_CORES = 2
NUM_SUBCORES = 16
NUM_WORKERS = NUM_CORES * NUM_SUBCORES

B_PER_W = 512
CHUNK = 256
N_CHUNKS = B_PER_W // CHUNK
DIM = 64
LANES = 16


def _glove_body(idx_hbm, wi_hbm, wj_hbm, out_hbm,
                idx_sh, idx_s, rows_a, rows_b, sem_a, sem_b):
    wid = lax.axis_index("s") * NUM_CORES + lax.axis_index("c")
    base = wid * B_PER_W
    pltpu.sync_copy(idx_hbm.at[pl.ds(base, B_PER_W)], idx_sh.at[wid])
    pltpu.sync_copy(idx_sh.at[wid], idx_s)

    for ch in range(N_CHUNKS):
        def fire(i, _):
            r = idx_s[ch * CHUNK + i]
            t = lax.shift_right_logical(r, 3)
            s = lax.rem(r, 8)
            pltpu.async_copy(wi_hbm.at[t, s], rows_a.at[i], sem_a)
            pltpu.async_copy(wj_hbm.at[t, s], rows_b.at[i], sem_b)
            return 0

        lax.fori_loop(0, CHUNK, fire, 0)

        dummy = out_hbm.at[pl.ds(base + ch * CHUNK, CHUNK)]
        pltpu.make_async_copy(dummy, rows_a, sem_a).wait()
        pltpu.make_async_copy(dummy, rows_b, sem_b).wait()

        def add(i, _):
            for c in range(DIM // LANES):
                sl = pl.ds(c * LANES, LANES)
                rows_a[i, sl] = rows_a[i, sl] + rows_b[i, sl]
            return 0

        lax.fori_loop(0, CHUNK, add, 0)

        pltpu.sync_copy(rows_a, out_hbm.at[pl.ds(base + ch * CHUNK, CHUNK)])


@jax.jit
def kernel(indices, wi, wj):
    batch = indices.shape[0]
    num_emb, dim = wi.shape
    assert batch == NUM_WORKERS * B_PER_W and dim == DIM

    wi = wi.reshape(num_emb // 8, 8, dim)
    wj = wj.reshape(num_emb // 8, 8, dim)
    mesh = plsc.VectorSubcoreMesh(core_axis_name="c", subcore_axis_name="s")
    run = pl.kernel(
        _glove_body,
        out_type=jax.ShapeDtypeStruct((batch, dim), jnp.float32),
        mesh=mesh,
        scratch_types=[
            pltpu.VMEM_SHARED((NUM_WORKERS, B_PER_W), jnp.int32),
            pltpu.SMEM((B_PER_W,), jnp.int32),
            pltpu.VMEM((CHUNK, DIM), jnp.float32),
            pltpu.VMEM((CHUNK, DIM), jnp.float32),
            pltpu.SemaphoreType.DMA,
            pltpu.SemaphoreType.DMA,
        ],
    )
    return run(indices.astype(jnp.int32), wi, wj)
